# baseline (speedup 1.0000x reference)
import functools

import jax
import jax.numpy as jnp
from jax import lax
from jax.experimental import pallas as pl
from jax.experimental.pallas import tpu as pltpu
from jax.experimental.pallas import tpu_sc as plsc

N = 10000
E = 320000
D = 128
DP = 144
NC, NS = 2, 16
NW = NC * NS
K = 128
CHUNKS = 80
EPW = CHUNKS * K
EP = NW * EPW
NR = 10240
RPS = NR // NS
ZR = 16
PAD_DST = N + 8


def _neg_log_table(x):
    def body(x_ref, o_ref):
        o_ref[...] = -jnp.log(x_ref[...])
    return pl.pallas_call(
        body, out_shape=jax.ShapeDtypeStruct((N, D), jnp.float32))(x)


def _combine(x, part):
    def body(x_ref, p_ref, o_ref):
        s = p_ref[0] + p_ref[1]
        sv = s[0:N, :]
        s_feat = sv[:, 0:D]
        deg = sv[:, D:DP].sum(axis=1)
        num = (x_ref[...] * s_feat).sum(axis=1)
        o_ref[...] = jnp.where(deg > 0, num / deg, 0.0)[:, None]
    return pl.pallas_call(
        body, out_shape=jax.ShapeDtypeStruct((N, 1), jnp.float32))(x, part)


def _make_sc_scatter():
    mesh = plsc.VectorSubcoreMesh(core_axis_name="c", subcore_axis_name="s")

    @functools.partial(
        pl.kernel,
        out_type=jax.ShapeDtypeStruct((NC, NR, DP), jnp.float32),
        mesh=mesh,
        compiler_params=pltpu.CompilerParams(use_tc_tiling_on_sc=False),
        scratch_types=[
            pltpu.VMEM((2, K), jnp.int32),
            pltpu.VMEM((2, K), jnp.int32),
            pltpu.VMEM((K, DP), jnp.float32),
            pltpu.VMEM((K, DP), jnp.float32),
            pltpu.VMEM_SHARED((NR, DP), jnp.float32),
            pltpu.SemaphoreType.DMA,
            pltpu.SemaphoreType.DMA,
            pltpu.SemaphoreType.DMA,
            pltpu.SemaphoreType.DMA,
        ],
    )
    def sc_scatter(lp_hbm, edges_hbm, part_hbm,
                   idx0, idx1, rows0, rows1, acc_sh,
                   semi0, semi1, semg0, semg1):
        cid = lax.axis_index("c")
        sid = lax.axis_index("s")
        wid = sid * NC + cid
        idxb = (idx0, idx1)
        rows = (rows0, rows1)
        semi = (semi0, semi1)
        semg = (semg0, semg1)

        def zrow(r, carry):
            for c9 in range(DP // 16):
                rows0[r, pl.ds(c9 * 16, 16)] = jnp.zeros((16,), jnp.float32)
            return carry
        lax.fori_loop(0, ZR, zrow, 0)

        def zcp(i, carry):
            pltpu.sync_copy(rows0.at[pl.ds(0, ZR)],
                            acc_sh.at[pl.ds(sid * RPS + i * ZR, ZR)])
            return carry
        lax.fori_loop(0, RPS // ZR, zcp, 0)
        plsc.subcore_barrier()

        def start_idx(c, b):
            pltpu.async_copy(edges_hbm.at[wid, c], idxb[b], semi[b])

        def wait_idx(b):
            pltpu.make_async_copy(edges_hbm.at[wid, 0], idxb[b],
                                  semi[b]).wait()

        def start_gather(b):
            pltpu.async_copy(lp_hbm.at[idxb[b].at[0]], rows[b], semg[b])

        def wait_gather(b):
            pltpu.make_async_copy(lp_hbm.at[idxb[b].at[0]], rows[b],
                                  semg[b]).wait()

        def scatter(b):
            pltpu.sync_copy(rows[b], acc_sh.at[idxb[b].at[1]], add=True)

        start_idx(0, 0)
        start_idx(1, 1)
        wait_idx(0)
        start_gather(0)

        def pair(g, carry):
            for b in (0, 1):
                c = 2 * g + b
                nb = 1 - b
                wait_gather(b)
                wait_idx(nb)
                start_gather(nb)
                scatter(b)
                start_idx(c + 2, b)
            return carry
        lax.fori_loop(0, CHUNKS // 2 - 1, pair, 0)

        wait_gather(0)
        wait_idx(1)
        start_gather(1)
        scatter(0)
        wait_gather(1)
        scatter(1)
        plsc.subcore_barrier()

        pltpu.sync_copy(acc_sh.at[pl.ds(sid * RPS, RPS)],
                        part_hbm.at[cid, pl.ds(sid * RPS, RPS)])

    return sc_scatter


_sc_scatter = _make_sc_scatter()


def kernel(x, edge_index):
    src = jnp.pad(edge_index[0], (0, EP - E)).reshape(NW, CHUNKS, 1, K)
    dst = jnp.pad(edge_index[1], (0, EP - E),
                  constant_values=PAD_DST).reshape(NW, CHUNKS, 1, K)
    edges = jnp.concatenate([src, dst], axis=2)

    logt = _neg_log_table(x)
    table = jnp.concatenate(
        [logt, jnp.ones((N, 1), jnp.float32), jnp.zeros((N, DP - D - 1),
                                                        jnp.float32)], axis=1)
    part = _sc_scatter(table, edges)
    return _combine(x, part).reshape(N)


# speedup vs baseline: 1.6834x; 1.6834x over previous
import functools

import jax
import jax.numpy as jnp
from jax import lax
from jax.experimental import pallas as pl
from jax.experimental.pallas import tpu as pltpu
from jax.experimental.pallas import tpu_sc as plsc

N = 10000
E = 320000
D = 128
DP = 144
NC, NS = 2, 16
NW = NC * NS
K = 128
CH0 = 118
CH1 = 40
TCH = NS * (CH0 + CH1)
EP = TCH * K
NR = 10240
RPS = NR // NS
ZR = 16
PAD_DST = N + 8


def _neg_log_table(x):
    def body(x_ref, o_ref):
        o_ref[...] = -jnp.log(x_ref[...])
    return pl.pallas_call(
        body, out_shape=jax.ShapeDtypeStruct((N, D), jnp.float32))(x)


def _combine(x, part):
    def body(x_ref, p_ref, o_ref):
        s = p_ref[0] + p_ref[1]
        sv = s[0:N, :]
        s_feat = sv[:, 0:D]
        deg = sv[:, D:DP].sum(axis=1)
        num = (x_ref[...] * s_feat).sum(axis=1)
        o_ref[...] = jnp.where(deg > 0, num / deg, 0.0)[:, None]
    return pl.pallas_call(
        body, out_shape=jax.ShapeDtypeStruct((N, 1), jnp.float32))(x, part)


def _make_sc_scatter():
    mesh = plsc.VectorSubcoreMesh(core_axis_name="c", subcore_axis_name="s")

    @functools.partial(
        pl.kernel,
        out_type=jax.ShapeDtypeStruct((NC, NR, DP), jnp.float32),
        mesh=mesh,
        compiler_params=pltpu.CompilerParams(use_tc_tiling_on_sc=False),
        scratch_types=[
            pltpu.VMEM((2, K), jnp.int32),
            pltpu.VMEM((2, K), jnp.int32),
            pltpu.VMEM((K, DP), jnp.float32),
            pltpu.VMEM((K, DP), jnp.float32),
            pltpu.VMEM_SHARED((NR, DP), jnp.float32),
            pltpu.SemaphoreType.DMA,
            pltpu.SemaphoreType.DMA,
            pltpu.SemaphoreType.DMA,
            pltpu.SemaphoreType.DMA,
        ],
    )
    def sc_scatter(lp_hbm, edges_hbm, part_hbm,
                   idx0, idx1, rows0, rows1, acc_sh,
                   semi0, semi1, semg0, semg1):
        cid = lax.axis_index("c")
        sid = lax.axis_index("s")
        ch = jnp.where(cid == 0, CH0, CH1)
        base = jnp.where(cid == 0, sid * CH0, NS * CH0 + sid * CH1)
        idxb = (idx0, idx1)
        rows = (rows0, rows1)
        semi = (semi0, semi1)
        semg = (semg0, semg1)

        def zrow(r, carry):
            for c9 in range(DP // 16):
                rows0[r, pl.ds(c9 * 16, 16)] = jnp.zeros((16,), jnp.float32)
            return carry
        lax.fori_loop(0, ZR, zrow, 0)

        def zcp(i, carry):
            pltpu.sync_copy(rows0.at[pl.ds(0, ZR)],
                            acc_sh.at[pl.ds(sid * RPS + i * ZR, ZR)])
            return carry
        lax.fori_loop(0, RPS // ZR, zcp, 0)
        plsc.subcore_barrier()

        def start_idx(c, b):
            pltpu.async_copy(edges_hbm.at[base + c], idxb[b], semi[b])

        def wait_idx(b):
            pltpu.make_async_copy(edges_hbm.at[base], idxb[b],
                                  semi[b]).wait()

        def start_gather(b):
            pltpu.async_copy(lp_hbm.at[idxb[b].at[0]], rows[b], semg[b])

        def wait_gather(b):
            pltpu.make_async_copy(lp_hbm.at[idxb[b].at[0]], rows[b],
                                  semg[b]).wait()

        def scatter(b):
            pltpu.sync_copy(rows[b], acc_sh.at[idxb[b].at[1]], add=True)

        start_idx(0, 0)
        start_idx(1, 1)
        wait_idx(0)
        start_gather(0)

        def pair(g, carry):
            for b in (0, 1):
                c = 2 * g + b
                nb = 1 - b
                wait_gather(b)
                wait_idx(nb)
                start_gather(nb)
                scatter(b)
                start_idx(c + 2, b)
            return carry
        lax.fori_loop(0, ch // 2 - 1, pair, 0)

        wait_gather(0)
        wait_idx(1)
        start_gather(1)
        scatter(0)
        wait_gather(1)
        scatter(1)
        plsc.subcore_barrier()

        pltpu.sync_copy(acc_sh.at[pl.ds(sid * RPS, RPS)],
                        part_hbm.at[cid, pl.ds(sid * RPS, RPS)])

    return sc_scatter


_sc_scatter = _make_sc_scatter()


def kernel(x, edge_index):
    src = jnp.pad(edge_index[0], (0, EP - E)).reshape(TCH, 1, K)
    dst = jnp.pad(edge_index[1], (0, EP - E),
                  constant_values=PAD_DST).reshape(TCH, 1, K)
    edges = jnp.concatenate([src, dst], axis=1)

    logt = _neg_log_table(x)
    table = jnp.concatenate(
        [logt, jnp.ones((N, 1), jnp.float32), jnp.zeros((N, DP - D - 1),
                                                        jnp.float32)], axis=1)
    part = _sc_scatter(table, edges)
    return _combine(x, part).reshape(N)


# speedup vs baseline: 2.1124x; 1.2549x over previous
import functools

import jax
import jax.numpy as jnp
from jax import lax
from jax.experimental import pallas as pl
from jax.experimental.pallas import tpu as pltpu
from jax.experimental.pallas import tpu_sc as plsc

N = 10000
E = 320000
D = 128
DP = 144
NC, NS = 2, 16
K = 128
TCH = E // K
T0 = 14 * 118 + 2 * 120
NR = 10240
RPS = NR // NS
ZR = 16


def _neg_log_table(x):
    def body(x_ref, o_ref):
        o_ref[...] = jnp.concatenate(
            [-jnp.log(x_ref[...]),
             jnp.ones((N, 1), jnp.float32),
             jnp.zeros((N, DP - D - 1), jnp.float32)], axis=1)
    return pl.pallas_call(
        body, out_shape=jax.ShapeDtypeStruct((N, DP), jnp.float32))(x)


def _combine(x, part):
    def body(x_ref, p_ref, o_ref):
        s = p_ref[0] + p_ref[1]
        sv = s[0:N, :]
        s_feat = sv[:, 0:D]
        deg = sv[:, D:DP].sum(axis=1)
        num = (x_ref[...] * s_feat).sum(axis=1)
        o_ref[...] = jnp.where(deg > 0, num / deg, 0.0)[:, None]
    return pl.pallas_call(
        body, out_shape=jax.ShapeDtypeStruct((N, 1), jnp.float32))(x, part)


def _make_sc_scatter():
    mesh = plsc.VectorSubcoreMesh(core_axis_name="c", subcore_axis_name="s")

    @functools.partial(
        pl.kernel,
        out_type=jax.ShapeDtypeStruct((NC, NR, DP), jnp.float32),
        mesh=mesh,
        compiler_params=pltpu.CompilerParams(use_tc_tiling_on_sc=False),
        scratch_types=[
            pltpu.VMEM((K,), jnp.int32),
            pltpu.VMEM((K,), jnp.int32),
            pltpu.VMEM((K,), jnp.int32),
            pltpu.VMEM((K,), jnp.int32),
            pltpu.VMEM((K, DP), jnp.float32),
            pltpu.VMEM((K, DP), jnp.float32),
            pltpu.VMEM_SHARED((NR, DP), jnp.float32),
            pltpu.SemaphoreType.DMA,
            pltpu.SemaphoreType.DMA,
            pltpu.SemaphoreType.DMA,
            pltpu.SemaphoreType.DMA,
        ],
    )
    def sc_scatter(ei_hbm, lp_hbm, part_hbm,
                   src0, src1, dst0, dst1, rows0, rows1, acc_sh,
                   semi0, semi1, semg0, semg1):
        cid = lax.axis_index("c")
        sid = lax.axis_index("s")
        cnt = jnp.where(cid == 0, jnp.where(sid >= 14, 120, 118), 38)
        base = jnp.where(cid == 0,
                         118 * sid + 2 * jnp.maximum(sid - 14, 0),
                         T0 + 38 * sid)
        srcb = (src0, src1)
        dstb = (dst0, dst1)
        rows = (rows0, rows1)
        semi = (semi0, semi1)
        semg = (semg0, semg1)

        def zrow(r, carry):
            for c9 in range(DP // 16):
                rows0[r, pl.ds(c9 * 16, 16)] = jnp.zeros((16,), jnp.float32)
            return carry
        lax.fori_loop(0, ZR, zrow, 0)

        def zcp(i, carry):
            pltpu.sync_copy(rows0.at[pl.ds(0, ZR)],
                            acc_sh.at[pl.ds(sid * RPS + i * ZR, ZR)])
            return carry
        lax.fori_loop(0, RPS // ZR, zcp, 0)
        plsc.subcore_barrier()

        def start_idx(c, b):
            off = (base + c) * K
            pltpu.async_copy(ei_hbm.at[0, pl.ds(off, K)], srcb[b], semi[b])
            pltpu.async_copy(ei_hbm.at[1, pl.ds(off, K)], dstb[b], semi[b])

        def wait_idx(b):
            pltpu.make_async_copy(ei_hbm.at[0, pl.ds(0, K)], srcb[b],
                                  semi[b]).wait()
            pltpu.make_async_copy(ei_hbm.at[1, pl.ds(0, K)], dstb[b],
                                  semi[b]).wait()

        def start_gather(b):
            pltpu.async_copy(lp_hbm.at[srcb[b]], rows[b], semg[b])

        def wait_gather(b):
            pltpu.make_async_copy(lp_hbm.at[srcb[b]], rows[b],
                                  semg[b]).wait()

        def scatter(b):
            pltpu.sync_copy(rows[b], acc_sh.at[dstb[b]], add=True)

        start_idx(0, 0)
        start_idx(1, 1)
        wait_idx(0)
        start_gather(0)

        def pair(g, carry):
            for b in (0, 1):
                c = 2 * g + b
                nb = 1 - b
                wait_gather(b)
                wait_idx(nb)
                start_gather(nb)
                scatter(b)
                start_idx(c + 2, b)
            return carry
        lax.fori_loop(0, cnt // 2 - 1, pair, 0)

        wait_gather(0)
        wait_idx(1)
        start_gather(1)
        scatter(0)
        wait_gather(1)
        scatter(1)
        plsc.subcore_barrier()

        pltpu.sync_copy(acc_sh.at[pl.ds(sid * RPS, RPS)],
                        part_hbm.at[cid, pl.ds(sid * RPS, RPS)])

    return sc_scatter


_sc_scatter = _make_sc_scatter()


def kernel(x, edge_index):
    table = _neg_log_table(x)
    part = _sc_scatter(edge_index, table)
    return _combine(x, part).reshape(N)


# speedup vs baseline: 2.5697x; 1.2165x over previous
import functools

import jax
import jax.numpy as jnp
from jax import lax
from jax.experimental import pallas as pl
from jax.experimental.pallas import tpu as pltpu
from jax.experimental.pallas import tpu_sc as plsc

N = 10000
E = 320000
D = 128
DP = 144
NC, NS = 2, 16
K = 128
TCH = E // K
T0 = 14 * 84 + 2 * 86
NR = 10240
RPS = NR // NS
ZR = 16


def _neg_log_table(x):
    def body(x_ref, o_ref):
        o_ref[...] = jnp.concatenate(
            [-jnp.log(x_ref[...]),
             jnp.ones((N, 1), jnp.float32),
             jnp.zeros((N, DP - D - 1), jnp.float32)], axis=1)
    return pl.pallas_call(
        body, out_shape=jax.ShapeDtypeStruct((N, DP), jnp.float32))(x)


def _combine(x, part):
    def body(x_ref, p_ref, o_ref):
        s = p_ref[0] + p_ref[1]
        sv = s[0:N, :]
        s_feat = sv[:, 0:D]
        deg = sv[:, D:DP].sum(axis=1)
        num = (x_ref[...] * s_feat).sum(axis=1)
        o_ref[...] = jnp.where(deg > 0, num / deg, 0.0)[:, None]
    return pl.pallas_call(
        body, out_shape=jax.ShapeDtypeStruct((N, 1), jnp.float32))(x, part)


def _make_sc_scatter():
    mesh = plsc.VectorSubcoreMesh(core_axis_name="c", subcore_axis_name="s")

    @functools.partial(
        pl.kernel,
        out_type=jax.ShapeDtypeStruct((NC, NR, DP), jnp.float32),
        mesh=mesh,
        compiler_params=pltpu.CompilerParams(use_tc_tiling_on_sc=False),
        scratch_types=[
            pltpu.VMEM((K,), jnp.int32),
            pltpu.VMEM((K,), jnp.int32),
            pltpu.VMEM((K,), jnp.int32),
            pltpu.VMEM((K,), jnp.int32),
            pltpu.VMEM((K, DP), jnp.float32),
            pltpu.VMEM((K, DP), jnp.float32),
            pltpu.VMEM_SHARED((NR, DP), jnp.float32),
            pltpu.SemaphoreType.DMA,
            pltpu.SemaphoreType.DMA,
            pltpu.SemaphoreType.DMA,
            pltpu.SemaphoreType.DMA,
        ],
    )
    def sc_scatter(ei_hbm, lp_hbm, part_hbm,
                   src0, src1, dst0, dst1, rows0, rows1, acc_sh,
                   semi0, semi1, semg0, semg1):
        cid = lax.axis_index("c")
        sid = lax.axis_index("s")
        cnt = jnp.where(cid == 0, jnp.where(sid >= 14, 86, 84), 72)
        base = jnp.where(cid == 0,
                         84 * sid + 2 * jnp.maximum(sid - 14, 0),
                         T0 + 72 * sid)
        srcb = (src0, src1)
        dstb = (dst0, dst1)
        rows = (rows0, rows1)
        semi = (semi0, semi1)
        semg = (semg0, semg1)

        def zrow(r, carry):
            for c9 in range(DP // 16):
                rows0[r, pl.ds(c9 * 16, 16)] = jnp.zeros((16,), jnp.float32)
            return carry
        lax.fori_loop(0, ZR, zrow, 0)

        def zcp(i, carry):
            pltpu.sync_copy(rows0.at[pl.ds(0, ZR)],
                            acc_sh.at[pl.ds(sid * RPS + i * ZR, ZR)])
            return carry
        lax.fori_loop(0, RPS // ZR, zcp, 0)
        plsc.subcore_barrier()

        def start_idx(c, b):
            off = (base + c) * K
            pltpu.async_copy(ei_hbm.at[0, pl.ds(off, K)], srcb[b], semi[b])
            pltpu.async_copy(ei_hbm.at[1, pl.ds(off, K)], dstb[b], semi[b])

        def wait_idx(b):
            pltpu.make_async_copy(ei_hbm.at[0, pl.ds(0, K)], srcb[b],
                                  semi[b]).wait()
            pltpu.make_async_copy(ei_hbm.at[1, pl.ds(0, K)], dstb[b],
                                  semi[b]).wait()

        def start_gather(b):
            pltpu.async_copy(lp_hbm.at[srcb[b]], rows[b], semg[b])

        def wait_gather(b):
            pltpu.make_async_copy(lp_hbm.at[srcb[b]], rows[b],
                                  semg[b]).wait()

        def scatter(b):
            pltpu.sync_copy(rows[b], acc_sh.at[dstb[b]], add=True)

        start_idx(0, 0)
        start_idx(1, 1)
        wait_idx(0)
        start_gather(0)

        def pair(g, carry):
            for b in (0, 1):
                c = 2 * g + b
                nb = 1 - b
                wait_gather(b)
                wait_idx(nb)
                start_gather(nb)
                scatter(b)
                start_idx(c + 2, b)
            return carry
        lax.fori_loop(0, cnt // 2 - 1, pair, 0)

        wait_gather(0)
        wait_idx(1)
        start_gather(1)
        scatter(0)
        wait_gather(1)
        scatter(1)
        plsc.subcore_barrier()

        pltpu.sync_copy(acc_sh.at[pl.ds(sid * RPS, RPS)],
                        part_hbm.at[cid, pl.ds(sid * RPS, RPS)])

    return sc_scatter


_sc_scatter = _make_sc_scatter()


def kernel(x, edge_index):
    table = _neg_log_table(x)
    part = _sc_scatter(edge_index, table)
    return _combine(x, part).reshape(N)


# speedup vs baseline: 2.9455x; 1.1462x over previous
import functools

import jax
import jax.numpy as jnp
from jax import lax
from jax.experimental import pallas as pl
from jax.experimental.pallas import tpu as pltpu
from jax.experimental.pallas import tpu_sc as plsc

N = 10000
E = 320000
D = 128
NC, NS = 2, 16
K = 128
TCH = E // K
T0 = 14 * 78 + 2 * 80
NR = 10240
RPS = NR // NS
ZR = 16


def _neg_log_table(x):
    def body(x_ref, o_ref):
        o_ref[...] = -jnp.log(x_ref[...])
    return pl.pallas_call(
        body, out_shape=jax.ShapeDtypeStruct((N, D), jnp.float32))(x)


def _combine(x, part, degp):
    def body(x_ref, p_ref, d_ref, o_ref):
        s = p_ref[0] + p_ref[1]
        deg = (d_ref[0] + d_ref[1])[0:N, :].sum(axis=1)
        num = (x_ref[...] * s[0:N, :]).sum(axis=1)
        o_ref[...] = jnp.where(deg > 0, num / deg, 0.0)[:, None]
    return pl.pallas_call(
        body, out_shape=jax.ShapeDtypeStruct((N, 1), jnp.float32))(
            x, part, degp)


def _make_sc_scatter():
    mesh = plsc.VectorSubcoreMesh(core_axis_name="c", subcore_axis_name="s")

    @functools.partial(
        pl.kernel,
        out_type=(jax.ShapeDtypeStruct((NC, NR, D), jnp.float32),
                  jax.ShapeDtypeStruct((NC, NR, 16), jnp.float32)),
        mesh=mesh,
        compiler_params=pltpu.CompilerParams(use_tc_tiling_on_sc=False),
        scratch_types=[
            pltpu.VMEM((K,), jnp.int32),
            pltpu.VMEM((K,), jnp.int32),
            pltpu.VMEM((K,), jnp.int32),
            pltpu.VMEM((K,), jnp.int32),
            pltpu.VMEM((K, D), jnp.float32),
            pltpu.VMEM((K, D), jnp.float32),
            pltpu.VMEM((K, 16), jnp.float32),
            pltpu.VMEM_SHARED((NR, D), jnp.float32),
            pltpu.VMEM_SHARED((NR, 16), jnp.float32),
            pltpu.SemaphoreType.DMA,
            pltpu.SemaphoreType.DMA,
            pltpu.SemaphoreType.DMA,
            pltpu.SemaphoreType.DMA,
        ],
    )
    def sc_scatter(src_hbm, dst_hbm, lp_hbm, part_hbm, deg_hbm,
                   src0, src1, dst0, dst1, rows0, rows1, ones_v, acc_sh,
                   deg_sh, semi0, semi1, semg0, semg1):
        cid = lax.axis_index("c")
        sid = lax.axis_index("s")
        cnt = jnp.where(cid == 0, jnp.where(sid >= 14, 80, 78), 78)
        base = jnp.where(cid == 0,
                         78 * sid + 2 * jnp.maximum(sid - 14, 0),
                         T0 + 78 * sid)
        srcb = (src0, src1)
        dstb = (dst0, dst1)
        rows = (rows0, rows1)
        semi = (semi0, semi1)
        semg = (semg0, semg1)
        zeros16 = jnp.zeros((16,), jnp.float32)
        ones16 = jnp.ones((16,), jnp.float32)

        one0 = jnp.where(lax.iota(jnp.int32, 16) == 0, 1.0, 0.0)

        def fones(r, carry):
            ones_v[r, pl.ds(0, 16)] = one0
            return carry
        lax.fori_loop(0, K, fones, 0)

        def zrow(r, carry):
            for c9 in range(D // 16):
                rows0[r, pl.ds(c9 * 16, 16)] = zeros16
            return carry
        lax.fori_loop(0, ZR, zrow, 0)

        def zcp(i, carry):
            pltpu.sync_copy(rows0.at[pl.ds(0, ZR)],
                            acc_sh.at[pl.ds(sid * RPS + i * ZR, ZR)])
            return carry
        lax.fori_loop(0, RPS // ZR, zcp, 0)

        def zcpd(i, carry):
            pltpu.sync_copy(rows0.at[pl.ds(0, ZR), pl.ds(0, 16)],
                            deg_sh.at[pl.ds(sid * RPS + i * ZR, ZR)])
            return carry
        lax.fori_loop(0, RPS // ZR, zcpd, 0)
        plsc.subcore_barrier()

        def start_idx(c, b):
            off = (base + c) * K
            pltpu.async_copy(src_hbm.at[pl.ds(off, K)], srcb[b], semi[b])
            pltpu.async_copy(dst_hbm.at[pl.ds(off, K)], dstb[b], semi[b])

        def wait_idx(b):
            pltpu.make_async_copy(src_hbm.at[pl.ds(0, K)], srcb[b],
                                  semi[b]).wait()
            pltpu.make_async_copy(dst_hbm.at[pl.ds(0, K)], dstb[b],
                                  semi[b]).wait()

        def start_gather(b):
            pltpu.async_copy(lp_hbm.at[srcb[b]], rows[b], semg[b])

        def wait_gather(b):
            pltpu.make_async_copy(lp_hbm.at[srcb[b]], rows[b],
                                  semg[b]).wait()

        def scatter(b):
            pltpu.sync_copy(rows[b], acc_sh.at[dstb[b]], add=True)

        def scatter_deg(b):
            pltpu.sync_copy(ones_v, deg_sh.at[dstb[b]], add=True)

        start_idx(0, 0)
        start_idx(1, 1)
        wait_idx(0)
        start_gather(0)

        def pair(g, carry):
            for b in (0, 1):
                c = 2 * g + b
                nb = 1 - b
                wait_gather(b)
                wait_idx(nb)
                start_gather(nb)
                scatter(b)
                scatter_deg(b)
                start_idx(c + 2, b)
            return carry
        lax.fori_loop(0, cnt // 2 - 1, pair, 0)

        wait_gather(0)
        wait_idx(1)
        start_gather(1)
        scatter(0)
        scatter_deg(0)
        wait_gather(1)
        scatter(1)
        scatter_deg(1)
        plsc.subcore_barrier()

        pltpu.sync_copy(acc_sh.at[pl.ds(sid * RPS, RPS)],
                        part_hbm.at[cid, pl.ds(sid * RPS, RPS)])
        pltpu.sync_copy(deg_sh.at[pl.ds(sid * RPS, RPS)],
                        deg_hbm.at[cid, pl.ds(sid * RPS, RPS)])

    return sc_scatter


_sc_scatter = _make_sc_scatter()


def kernel(x, edge_index):
    table = _neg_log_table(x)
    part, degp = _sc_scatter(edge_index[0], edge_index[1], table)
    return _combine(x, part, degp).reshape(N)
